# baseline (speedup 1.0000x reference)
import functools

import jax
import jax.numpy as jnp
from jax.experimental import pallas as pl
from jax.experimental.pallas import tpu as pltpu
from jax.experimental.pallas import tpu_sc as plsc

AXIS = 4


def _sc_gather(table, idx, window):
    n_steps, _ = idx.shape
    d = table.shape[1]
    mesh = plsc.VectorSubcoreMesh(core_axis_name="core", subcore_axis_name="subcore")

    @functools.partial(
        pl.kernel,
        out_type=jax.ShapeDtypeStruct((n_steps * window, d), table.dtype),
        mesh=mesh,
    )
    def gather_kernel(x_hbm, i_hbm, o_hbm):
        def body(i_vmem, o_vmem):
            pltpu.sync_copy(x_hbm.at[i_vmem.at[0]], o_vmem)

        pltpu.emit_pipeline(
            body,
            grid=(n_steps,),
            in_specs=[pl.BlockSpec((1, window), index_map=lambda i: (i, 0))],
            out_specs=[pl.BlockSpec((window, d), index_map=lambda i: (i, 0))],
            core_axis_name=("core", "subcore"),
            dimension_semantics=(pltpu.PARALLEL,),
        )(i_hbm, o_hbm)

    return gather_kernel(table, idx)


def _tc_body(
    node_ref, nei_ref, edge_ref, cf_ref, angle_ref,
    w_self_ref, wc_node_ref, wc_nei_ref, wc_edge_ref, w_sym_ref,
    b_self_ref, b_c_ref, b_sym_ref,
    r_n0_ref, r_n1_ref, r_n2_ref, r_e0_ref,
    nout_ref, eout_ref, aout_ref,
    copy_sem,
):
    nblocks = pl.num_programs(0)
    pid = pl.program_id(0)

    @pl.when(pid == 0)
    def _():
        pltpu.make_async_copy(angle_ref, aout_ref, copy_sem).start()

    nnei, b, ndim = nei_ref.shape
    edim = edge_ref.shape[2]
    e = b * nnei
    inv_nnei = jnp.bfloat16(1.0 / nnei)

    node = node_ref[...]
    node_bf = node.astype(jnp.bfloat16)
    nei = nei_ref[...].astype(jnp.bfloat16)
    edge = edge_ref[...]
    edge_bf = edge.astype(jnp.bfloat16)
    cf = cf_ref[...]

    silu = jax.nn.silu

    node_self = silu(jnp.dot(node_bf, w_self_ref[...],
                             preferred_element_type=jnp.float32) + b_self_ref[...])

    nei_f = nei.reshape(e, ndim)
    edge_f = edge_bf.reshape(e, edim)
    pre_n = jnp.dot(node_bf, wc_node_ref[...],
                    preferred_element_type=jnp.float32).astype(jnp.bfloat16)
    pre_ne = (
        jnp.dot(nei_f, wc_nei_ref[:, :ndim],
                preferred_element_type=jnp.float32).astype(jnp.bfloat16)
        + jnp.dot(edge_f, wc_edge_ref[:, :ndim],
                  preferred_element_type=jnp.float32).astype(jnp.bfloat16)
    ).reshape(nnei, b, ndim)
    pre_ee = (
        jnp.dot(nei_f, wc_nei_ref[:, ndim:],
                preferred_element_type=jnp.float32).astype(jnp.bfloat16)
        + jnp.dot(edge_f, wc_edge_ref[:, ndim:],
                  preferred_element_type=jnp.float32).astype(jnp.bfloat16)
    ).reshape(nnei, b, edim)
    act_ne = silu(pre_ne + pre_n[None, :, :ndim] + b_c_ref[:, :ndim][None])
    act_ee = silu(pre_ee + pre_n[None, :, ndim:] + b_c_ref[:, ndim:][None])

    eout_ref[...] = edge + r_e0_ref[...][None] * act_ee.astype(jnp.float32)

    sw3 = cf[:, :, 3:4]
    csw = cf * sw3

    bfz = jnp.bfloat16(0)
    msg_acc = jnp.full((b, ndim), bfz)
    for i in range(nnei):
        msg_acc = msg_acc + act_ne[i] * sw3[i]
    msg = msg_acc * inv_nnei

    hgn = [jnp.full((b, ndim), bfz) for _ in range(3)]
    for i in range(nnei):
        nei_i = nei[i]
        for k in range(3):
            hgn[k] = hgn[k] + nei_i * csw[i, :, k:k + 1]
    hgn = [h * inv_nnei for h in hgn]

    hge = [jnp.full((b, edim), bfz) for _ in range(3)]
    for i in range(nnei):
        edge_i = edge_bf[i]
        for k in range(3):
            hge[k] = hge[k] + edge_i * csw[i, :, k:k + 1]
    hge = [h * inv_nnei for h in hge]

    sym_parts = []
    for a in range(AXIS):
        se = hge[0][:, a:a + 1] * hge[0]
        for k in range(1, 3):
            se = se + hge[k][:, a:a + 1] * hge[k]
        sym_parts.append(se)
    for a in range(AXIS):
        sn = hgn[0][:, a:a + 1] * hgn[0]
        for k in range(1, 3):
            sn = sn + hgn[k][:, a:a + 1] * hgn[k]
        sym_parts.append(sn)
    sym = jnp.concatenate(sym_parts, axis=-1)

    node_sym = silu(jnp.dot(sym, w_sym_ref[...],
                            preferred_element_type=jnp.float32) + b_sym_ref[...])

    nout_ref[...] = (
        node
        + r_n0_ref[...] * node_self
        + r_n1_ref[...] * node_sym
        + r_n2_ref[...] * msg.astype(jnp.float32)
    )

    @pl.when(pid == nblocks - 1)
    def _():
        pltpu.make_async_copy(angle_ref, aout_ref, copy_sem).wait()


def _tc_dense(node, nei_t, edge_t, cf_t, angle, w_self, wc_node, wc_nei,
              wc_edge, w_sym_p, b_self, b_c, b_sym, r_n0, r_n1, r_n2, r_e0,
              block):
    nloc, ndim = node.shape
    nnei, _, edim = edge_t.shape
    nf = wc_node.shape[1]
    grid = (nloc // block,)

    full = lambda shape: pl.BlockSpec(shape, lambda i: tuple(0 for _ in shape))
    hbm = pl.BlockSpec(memory_space=pl.ANY)
    out_shapes = (
        jax.ShapeDtypeStruct((nloc, ndim), jnp.float32),
        jax.ShapeDtypeStruct((nnei, nloc, edim), jnp.float32),
        jax.ShapeDtypeStruct(angle.shape, angle.dtype),
    )
    return pl.pallas_call(
        _tc_body,
        grid=grid,
        in_specs=[
            pl.BlockSpec((block, ndim), lambda i: (i, 0)),
            pl.BlockSpec((nnei, block, ndim), lambda i: (0, i, 0)),
            pl.BlockSpec((nnei, block, edim), lambda i: (0, i, 0)),
            pl.BlockSpec((nnei, block, 4), lambda i: (0, i, 0)),
            hbm,
            full((ndim, ndim)),
            full((ndim, nf)),
            full((ndim, nf)),
            full((edim, nf)),
            full((w_sym_p.shape[0], ndim)),
            full((1, ndim)),
            full((1, nf)),
            full((1, ndim)),
            full((1, ndim)),
            full((1, ndim)),
            full((1, ndim)),
            full((1, edim)),
        ],
        out_specs=[
            pl.BlockSpec((block, ndim), lambda i: (i, 0)),
            pl.BlockSpec((nnei, block, edim), lambda i: (0, i, 0)),
            hbm,
        ],
        out_shape=out_shapes,
        scratch_shapes=[pltpu.SemaphoreType.DMA],
    )(node, nei_t, edge_t, cf_t, angle, w_self, wc_node, wc_nei, wc_edge,
      w_sym_p, b_self, b_c, b_sym, r_n0, r_n1, r_n2, r_e0)


def _sym_perm(ndim, edim, axis):
    idx = []
    for a in range(axis):
        for d_ in range(edim):
            idx.append(d_ * axis + a)
    for a in range(axis):
        for d_ in range(ndim):
            idx.append(edim * axis + d_ * axis + a)
    return jnp.array(idx, dtype=jnp.int32)


def kernel(node_ebd_ext, edge_ebd, h2, angle_ebd, nlist, nlist_mask, sw,
           angle_nlist, angle_nlist_mask, a_sw, W_self, b_self, W_sym, b_sym,
           W_ne, b_ne, W_ee, b_ee, r_n0, r_n1, r_n2, r_e0):
    nb, nloc, nnei, edim = edge_ebd.shape
    ndim = node_ebd_ext.shape[-1]
    e_tot = nloc * nnei
    bf = jnp.bfloat16

    table = node_ebd_ext.reshape(-1, ndim)

    window = 400
    idx = jax.lax.optimization_barrier(
        nlist[0].T.reshape(e_tot // window, window).astype(jnp.int32))

    nei_t = _sc_gather(table, idx, window=window).reshape(nnei, nloc, ndim)

    node = node_ebd_ext[0, :nloc, :]
    edge_t = jnp.transpose(edge_ebd[0], (1, 0, 2))
    sw_m = (sw * nlist_mask.astype(sw.dtype))[0].T[:, :, None]
    cf_t = jnp.concatenate(
        [jnp.transpose(h2[0], (1, 0, 2)), sw_m], axis=-1).astype(bf)
    angle_flat = angle_ebd.reshape(nloc, -1)

    wc = jnp.concatenate([W_ne, W_ee], axis=1).astype(bf)
    wc_node = wc[:ndim]
    wc_nei = wc[ndim:2 * ndim]
    wc_edge = wc[2 * ndim:]
    b_c = jnp.concatenate([b_ne, b_ee]).reshape(1, -1).astype(bf)
    w_sym_p = W_sym[_sym_perm(ndim, edim, AXIS)].astype(bf)

    n_upd, e_upd_t, a_upd = _tc_dense(
        node, nei_t, edge_t, cf_t, angle_flat, W_self.astype(bf), wc_node,
        wc_nei, wc_edge, w_sym_p, b_self.reshape(1, -1), b_c,
        b_sym.reshape(1, -1), r_n0.reshape(1, -1), r_n1.reshape(1, -1),
        r_n2.reshape(1, -1), r_e0.reshape(1, -1), block=200,
    )

    n_updated = n_upd.reshape(nb, nloc, ndim)
    e_updated = jnp.transpose(e_upd_t, (1, 0, 2)).reshape(nb, nloc, nnei, edim)
    a_updated = a_upd.reshape(angle_ebd.shape)
    return n_updated, e_updated, a_updated


# speedup vs baseline: 5.5811x; 5.5811x over previous
import functools

import jax
import jax.numpy as jnp
from jax.experimental import pallas as pl
from jax.experimental.pallas import tpu as pltpu
from jax.experimental.pallas import tpu_sc as plsc

AXIS = 4


def _sc_gather(table, idx, window):
    n_steps, _ = idx.shape
    d = table.shape[1]
    mesh = plsc.VectorSubcoreMesh(core_axis_name="core", subcore_axis_name="subcore")

    @functools.partial(
        pl.kernel,
        out_type=jax.ShapeDtypeStruct((n_steps * window, d), table.dtype),
        mesh=mesh,
    )
    def gather_kernel(x_hbm, i_hbm, o_hbm):
        def body(i_vmem, o_vmem):
            pltpu.sync_copy(x_hbm.at[i_vmem.at[0]], o_vmem)

        pltpu.emit_pipeline(
            body,
            grid=(n_steps,),
            in_specs=[pl.BlockSpec((1, window), index_map=lambda i: (i, 0))],
            out_specs=[pl.BlockSpec((window, d), index_map=lambda i: (i, 0))],
            core_axis_name=("core", "subcore"),
            dimension_semantics=(pltpu.PARALLEL,),
        )(i_hbm, o_hbm)

    return gather_kernel(table, idx)


def _tc_body(
    node_ref, nei_ref, edge_ref, cf_ref, angle_ref,
    w_self_ref, wc_node_ref, wc_nei_ref, wc_edge_ref, w_sym_ref, sel_ref,
    b_self_ref, b_c_ref, b_sym_ref,
    r_n0_ref, r_n1_ref, r_n2_ref, r_e0_ref,
    nout_ref, eout_ref, aout_ref,
    copy_sem,
):
    nnei, b, ndim = nei_ref.shape
    edim = edge_ref.shape[2]
    e = b * nnei
    inv_nnei = jnp.bfloat16(1.0 / nnei)
    pid = pl.program_id(0)
    a_rows = aout_ref.shape[0]

    angle_dma = pltpu.make_async_copy(
        angle_ref.at[pl.ds(pid * a_rows, a_rows)], aout_ref, copy_sem)
    angle_dma.start()

    node = node_ref[...]
    node_bf = node.astype(jnp.bfloat16)
    nei = nei_ref[...].astype(jnp.bfloat16)
    edge = edge_ref[...]
    edge_bf = edge.astype(jnp.bfloat16)
    cbc = jnp.dot(cf_ref[...].reshape(e, 4), sel_ref[...],
                  preferred_element_type=jnp.float32).astype(jnp.bfloat16)
    cbc = cbc.reshape(nnei, b, 4 * ndim)

    silu = jax.nn.silu

    node_self = silu(jnp.dot(node_bf, w_self_ref[...],
                             preferred_element_type=jnp.float32) + b_self_ref[...])

    nei_f = nei.reshape(e, ndim)
    edge_f = edge_bf.reshape(e, edim)
    pre_n = jnp.dot(node_bf, wc_node_ref[...],
                    preferred_element_type=jnp.float32).astype(jnp.bfloat16)
    pre_ne = (
        jnp.dot(nei_f, wc_nei_ref[:, :ndim],
                preferred_element_type=jnp.float32).astype(jnp.bfloat16)
        + jnp.dot(edge_f, wc_edge_ref[:, :ndim],
                  preferred_element_type=jnp.float32).astype(jnp.bfloat16)
    ).reshape(nnei, b, ndim)
    pre_ee = (
        jnp.dot(nei_f, wc_nei_ref[:, ndim:],
                preferred_element_type=jnp.float32).astype(jnp.bfloat16)
        + jnp.dot(edge_f, wc_edge_ref[:, ndim:],
                  preferred_element_type=jnp.float32).astype(jnp.bfloat16)
    ).reshape(nnei, b, edim)
    act_ne = silu(pre_ne + pre_n[None, :, :ndim] + b_c_ref[:, :ndim][None])
    act_ee = silu(pre_ee + pre_n[None, :, ndim:] + b_c_ref[:, ndim:][None])

    eout_ref[...] = edge + r_e0_ref[...][None] * act_ee.astype(jnp.float32)

    bfz = jnp.bfloat16(0)
    msg_acc = jnp.full((b, ndim), bfz)
    for i in range(nnei):
        msg_acc = msg_acc + act_ne[i] * cbc[i, :, 3 * ndim:]
    msg = msg_acc * inv_nnei

    hgn = [jnp.full((b, ndim), bfz) for _ in range(3)]
    for i in range(nnei):
        nei_i = nei[i]
        for k in range(3):
            hgn[k] = hgn[k] + nei_i * cbc[i, :, k * ndim:(k + 1) * ndim]
    hgn = [h * inv_nnei for h in hgn]

    hge = [jnp.full((b, edim), bfz) for _ in range(3)]
    for i in range(nnei):
        edge_i = edge_bf[i]
        for k in range(3):
            hge[k] = hge[k] + edge_i * cbc[i, :, k * ndim:k * ndim + edim]
    hge = [h * inv_nnei for h in hge]

    sym_parts = []
    for a in range(AXIS):
        se = hge[0][:, a:a + 1] * hge[0]
        for k in range(1, 3):
            se = se + hge[k][:, a:a + 1] * hge[k]
        sym_parts.append(se)
    for a in range(AXIS):
        sn = hgn[0][:, a:a + 1] * hgn[0]
        for k in range(1, 3):
            sn = sn + hgn[k][:, a:a + 1] * hgn[k]
        sym_parts.append(sn)
    sym = jnp.concatenate(sym_parts, axis=-1)

    node_sym = silu(jnp.dot(sym, w_sym_ref[...],
                            preferred_element_type=jnp.float32) + b_sym_ref[...])

    nout_ref[...] = (
        node
        + r_n0_ref[...] * node_self
        + r_n1_ref[...] * node_sym
        + r_n2_ref[...] * msg.astype(jnp.float32)
    )

    angle_dma.wait()


def _tc_dense(node, nei_t, edge_t, cf_t, angle3, w_self, wc_node, wc_nei,
              wc_edge, w_sym_p, sel, b_self, b_c, b_sym, r_n0, r_n1, r_n2,
              r_e0, block):
    nloc, ndim = node.shape
    nnei, _, edim = edge_t.shape
    nf = wc_node.shape[1]
    grid = (nloc // block,)
    a_rows = angle3.shape[0] // (nloc // block)

    full = lambda shape: pl.BlockSpec(shape, lambda i: tuple(0 for _ in shape))
    out_shapes = (
        jax.ShapeDtypeStruct((nloc, ndim), jnp.float32),
        jax.ShapeDtypeStruct((nnei, nloc, edim), jnp.float32),
        jax.ShapeDtypeStruct(angle3.shape, angle3.dtype),
    )
    return pl.pallas_call(
        _tc_body,
        grid=grid,
        in_specs=[
            pl.BlockSpec((block, ndim), lambda i: (i, 0)),
            pl.BlockSpec((nnei, block, ndim), lambda i: (0, i, 0)),
            pl.BlockSpec((nnei, block, edim), lambda i: (0, i, 0)),
            pl.BlockSpec((nnei, block, 4), lambda i: (0, i, 0)),
            pl.BlockSpec(memory_space=pl.ANY),
            full((ndim, ndim)),
            full((ndim, nf)),
            full((ndim, nf)),
            full((edim, nf)),
            full((w_sym_p.shape[0], ndim)),
            full((4, 4 * ndim)),
            full((1, ndim)),
            full((1, nf)),
            full((1, ndim)),
            full((1, ndim)),
            full((1, ndim)),
            full((1, ndim)),
            full((1, edim)),
        ],
        out_specs=[
            pl.BlockSpec((block, ndim), lambda i: (i, 0)),
            pl.BlockSpec((nnei, block, edim), lambda i: (0, i, 0)),
            pl.BlockSpec((a_rows,) + angle3.shape[1:], lambda i: (i, 0, 0)),
        ],
        out_shape=out_shapes,
        scratch_shapes=[pltpu.SemaphoreType.DMA],
    )(node, nei_t, edge_t, cf_t, angle3, w_self, wc_node, wc_nei, wc_edge,
      w_sym_p, sel, b_self, b_c, b_sym, r_n0, r_n1, r_n2, r_e0)


def _sym_perm(ndim, edim, axis):
    idx = []
    for a in range(axis):
        for d_ in range(edim):
            idx.append(d_ * axis + a)
    for a in range(axis):
        for d_ in range(ndim):
            idx.append(edim * axis + d_ * axis + a)
    return jnp.array(idx, dtype=jnp.int32)


def kernel(node_ebd_ext, edge_ebd, h2, angle_ebd, nlist, nlist_mask, sw,
           angle_nlist, angle_nlist_mask, a_sw, W_self, b_self, W_sym, b_sym,
           W_ne, b_ne, W_ee, b_ee, r_n0, r_n1, r_n2, r_e0):
    nb, nloc, nnei, edim = edge_ebd.shape
    ndim = node_ebd_ext.shape[-1]
    e_tot = nloc * nnei
    bf = jnp.bfloat16

    table = node_ebd_ext.reshape(-1, ndim)
    window = 400
    idx = jax.lax.optimization_barrier(
        nlist[0].T.reshape(e_tot // window, window).astype(jnp.int32))

    nei_t = _sc_gather(table, idx, window=window).reshape(nnei, nloc, ndim)

    node = node_ebd_ext[0, :nloc, :]
    edge_t = jnp.transpose(edge_ebd[0], (1, 0, 2))
    sw_m = (sw * nlist_mask.astype(sw.dtype))[0].T[:, :, None]
    cf_t = jnp.concatenate(
        [jnp.transpose(h2[0], (1, 0, 2)) * sw_m, sw_m], axis=-1).astype(bf)
    a_shape = angle_ebd.shape
    angle3 = angle_ebd.reshape(nloc * a_shape[2], a_shape[3], a_shape[4])

    wc = jnp.concatenate([W_ne, W_ee], axis=1).astype(bf)
    wc_node = wc[:ndim]
    wc_nei = wc[ndim:2 * ndim]
    wc_edge = wc[2 * ndim:]
    b_c = jnp.concatenate([b_ne, b_ee]).reshape(1, -1).astype(bf)
    w_sym_p = W_sym[_sym_perm(ndim, edim, AXIS)].astype(bf)
    sel = jnp.repeat(jnp.eye(4, dtype=bf), ndim, axis=1)

    n_upd, e_upd_t, a_upd = _tc_dense(
        node, nei_t, edge_t, cf_t, angle3, W_self.astype(bf), wc_node,
        wc_nei, wc_edge, w_sym_p, sel, b_self.reshape(1, -1), b_c,
        b_sym.reshape(1, -1), r_n0.reshape(1, -1), r_n1.reshape(1, -1),
        r_n2.reshape(1, -1), r_e0.reshape(1, -1), block=200,
    )

    n_updated = n_upd.reshape(nb, nloc, ndim)
    e_updated = jnp.transpose(e_upd_t, (1, 0, 2)).reshape(nb, nloc, nnei, edim)
    a_updated = a_upd.reshape(a_shape)
    return n_updated, e_updated, a_updated
